# baseline (speedup 1.0000x reference)
import functools

import jax
import jax.numpy as jnp
from jax import lax
from jax.experimental import pallas as pl
from jax.experimental.pallas import tpu as pltpu
from jax.experimental.pallas import tpu_sc as plsc

NUM_PATCHES = 1024
D = 768
B = 64
NUM_WORKERS = 32
F_PER_W = D // NUM_WORKERS
LANES = 16
VECS_PER_ROW = D // LANES


def _emb_body(x_hbm, tok_hbm, pos_hbm, out_hbm, pos_v, idx_v, rows_v, sem):
    c = lax.axis_index("c")
    s = lax.axis_index("s")
    wid = s * 2 + c
    f0 = wid * F_PER_W

    pltpu.sync_copy(pos_hbm.at[pl.ds(f0, F_PER_W)], pos_v)

    def body(b, carry):
        pltpu.sync_copy(x_hbm.at[pl.ds(b * D + f0, F_PER_W)], idx_v)
        pltpu.async_copy(tok_hbm.at[idx_v], rows_v, sem).wait()
        for r in range(F_PER_W):
            def inner(j, c2):
                sl = pl.ds(j * LANES, LANES)
                plsc.addupdate(rows_v.at[r, sl], pos_v[r, sl])
                return c2
            lax.fori_loop(0, VECS_PER_ROW, inner, 0)
        pltpu.sync_copy(rows_v, out_hbm.at[pl.ds(b * D + f0, F_PER_W)])
        return carry

    lax.fori_loop(0, B, body, 0)


@jax.jit
def kernel(x, token_table, pos_table):
    mesh = plsc.VectorSubcoreMesh(core_axis_name="c", subcore_axis_name="s")
    out = pl.kernel(
        _emb_body,
        out_type=jax.ShapeDtypeStruct((B * D, D), jnp.float32),
        mesh=mesh,
        scratch_types=[
            pltpu.VMEM((F_PER_W, D), jnp.float32),
            pltpu.VMEM((F_PER_W,), jnp.int32),
            pltpu.VMEM((F_PER_W, D), jnp.float32),
            pltpu.SemaphoreType.DMA,
        ],
    )(x.reshape(-1), token_table, pos_table)
    return out.reshape(B, D, D)


# speedup vs baseline: 2.4305x; 2.4305x over previous
import functools

import jax
import jax.numpy as jnp
from jax import lax
from jax.experimental import pallas as pl
from jax.experimental.pallas import tpu as pltpu
from jax.experimental.pallas import tpu_sc as plsc

NUM_PATCHES = 1024
D = 768
B = 64
NUM_WORKERS = 32
F_PER_W = D // NUM_WORKERS
LANES = 16
VECS_PER_ROW = D // LANES


def _emb_body(x_hbm, tok_hbm, pos_hbm, out_hbm,
              pos_v, idx0, idx1, rows0, rows1, g0, g1, s0, s1):
    c = lax.axis_index("c")
    s = lax.axis_index("s")
    wid = s * 2 + c
    f0 = wid * F_PER_W

    idx = (idx0, idx1)
    rows = (rows0, rows1)
    gsem = (g0, g1)
    ssem = (s0, s1)

    pltpu.sync_copy(pos_hbm.at[pl.ds(f0, F_PER_W)], pos_v)

    def out_slice(bb):
        return out_hbm.at[pl.ds(bb * D + f0, F_PER_W)]

    pltpu.sync_copy(x_hbm.at[pl.ds(f0, F_PER_W)], idx0)
    pltpu.async_copy(tok_hbm.at[idx0], rows0, g0)

    def step(i, k):
        bb = 2 * i + k
        cur, nxt = rows[k], rows[1 - k]
        inxt = idx[1 - k]

        @pl.when(bb >= 1)
        def _():
            pltpu.make_async_copy(nxt, out_slice(bb - 1), ssem[1 - k]).wait()

        @pl.when(bb < B - 1)
        def _():
            pltpu.sync_copy(x_hbm.at[pl.ds((bb + 1) * D + f0, F_PER_W)], inxt)
            pltpu.async_copy(tok_hbm.at[inxt], nxt, gsem[1 - k])

        pltpu.make_async_copy(tok_hbm.at[idx[k]], cur, gsem[k]).wait()

        @pl.loop(0, F_PER_W)
        def _(r):
            for j in range(VECS_PER_ROW):
                sl = pl.ds(j * LANES, LANES)
                plsc.addupdate(cur.at[r, sl], pos_v[r, sl])

        pltpu.async_copy(cur, out_slice(bb), ssem[k])

    def body(i, carry):
        step(i, 0)
        step(i, 1)
        return carry

    lax.fori_loop(0, B // 2, body, 0)
    pltpu.make_async_copy(rows[1], out_slice(B - 1), ssem[1]).wait()


@jax.jit
def kernel(x, token_table, pos_table):
    mesh = plsc.VectorSubcoreMesh(core_axis_name="c", subcore_axis_name="s")
    out = pl.kernel(
        _emb_body,
        out_type=jax.ShapeDtypeStruct((B * D, D), jnp.float32),
        mesh=mesh,
        scratch_types=[
            pltpu.VMEM((F_PER_W, D), jnp.float32),
            pltpu.VMEM((F_PER_W,), jnp.int32),
            pltpu.VMEM((F_PER_W,), jnp.int32),
            pltpu.VMEM((F_PER_W, D), jnp.float32),
            pltpu.VMEM((F_PER_W, D), jnp.float32),
            pltpu.SemaphoreType.DMA,
            pltpu.SemaphoreType.DMA,
            pltpu.SemaphoreType.DMA,
            pltpu.SemaphoreType.DMA,
        ],
    )(x.reshape(-1), token_table, pos_table)
    return out.reshape(B, D, D)


# speedup vs baseline: 2.9044x; 1.1950x over previous
import jax
import jax.numpy as jnp
from jax import lax
from jax.experimental import pallas as pl
from jax.experimental.pallas import tpu as pltpu
from jax.experimental.pallas import tpu_sc as plsc

NUM_PATCHES = 1024
D = 768
B = 64
NUM_WORKERS = 32
F_PER_W = D // NUM_WORKERS
LANES = 16
VECS_PER_ROW = D // LANES
IDX_PER_W = B * F_PER_W


def _emb_body(x_hbm, tok_hbm, pos_hbm, out_hbm,
              pos_v, idx_all, rows0, rows1, g0, g1, s0, s1):
    c = lax.axis_index("c")
    s = lax.axis_index("s")
    wid = s * 2 + c
    f0 = wid * F_PER_W

    rows = (rows0, rows1)
    gsem = (g0, g1)
    ssem = (s0, s1)

    pltpu.sync_copy(pos_hbm.at[pl.ds(f0, F_PER_W)], pos_v)
    pltpu.sync_copy(x_hbm.at[pl.ds(wid * IDX_PER_W, IDX_PER_W)], idx_all)

    def idx_slice(bb):
        return idx_all.at[pl.ds(bb * F_PER_W, F_PER_W)]

    def out_slice(bb):
        return out_hbm.at[pl.ds(bb * D + f0, F_PER_W)]

    pltpu.async_copy(tok_hbm.at[idx_slice(0)], rows0, g0)

    def step(i, k):
        bb = 2 * i + k
        cur, nxt = rows[k], rows[1 - k]

        @pl.when(bb >= 1)
        def _():
            pltpu.make_async_copy(nxt, out_slice(bb - 1), ssem[1 - k]).wait()

        @pl.when(bb < B - 1)
        def _():
            pltpu.async_copy(tok_hbm.at[idx_slice(bb + 1)], nxt, gsem[1 - k])

        pltpu.make_async_copy(tok_hbm.at[idx_slice(bb)], cur, gsem[k]).wait()

        @pl.loop(0, F_PER_W)
        def _(r):
            for j in range(VECS_PER_ROW):
                sl = pl.ds(j * LANES, LANES)
                plsc.addupdate(cur.at[r, sl], pos_v[r, sl])

        pltpu.async_copy(cur, out_slice(bb), ssem[k])

    def body(i, carry):
        step(i, 0)
        step(i, 1)
        return carry

    lax.fori_loop(0, B // 2, body, 0)
    pltpu.make_async_copy(rows[1], out_slice(B - 1), ssem[1]).wait()


@jax.jit
def kernel(x, token_table, pos_table):
    xp = x.reshape(B, NUM_WORKERS, F_PER_W).transpose(1, 0, 2).reshape(-1)
    mesh = plsc.VectorSubcoreMesh(core_axis_name="c", subcore_axis_name="s")
    out = pl.kernel(
        _emb_body,
        out_type=jax.ShapeDtypeStruct((B * D, D), jnp.float32),
        mesh=mesh,
        scratch_types=[
            pltpu.VMEM((F_PER_W, D), jnp.float32),
            pltpu.VMEM((IDX_PER_W,), jnp.int32),
            pltpu.VMEM((F_PER_W, D), jnp.float32),
            pltpu.VMEM((F_PER_W, D), jnp.float32),
            pltpu.SemaphoreType.DMA,
            pltpu.SemaphoreType.DMA,
            pltpu.SemaphoreType.DMA,
            pltpu.SemaphoreType.DMA,
        ],
    )(xp, token_table, pos_table)
    return out.reshape(B, D, D)
